# Initial kernel scaffold; baseline (speedup 1.0000x reference)
import jax
import jax.numpy as jnp
from jax.experimental import pallas as pl


def kernel(x, edge_index, batch, W1, b1, W2, b2, W3, b3, lin_W, lin_b):
    raise NotImplementedError("write your pallas kernel here")


# speedup vs baseline: 10.6062x; 10.6062x over previous
import functools

import jax
import jax.numpy as jnp
from jax import lax
from jax.experimental import pallas as pl
from jax.experimental.pallas import tpu as pltpu
from jax.experimental.pallas import tpu_sc as plsc

N = 100000
E = 3200000
G = 128
HID = 64
CW = 16
NC, NS = 2, 16
KR = 8
E_PAD = 3211264
ROWS = E_PAD // 128
NP = 102400
TPT = NP // NS
BLK = 1000
GRID = N // BLK


@functools.lru_cache(maxsize=None)
def _sc_pass(n_tables: int, split_edges: bool, no_gather: bool):
    tpc = 1 if split_edges else n_tables
    n_out = NC if split_edges else n_tables
    rpt = ROWS // (NC * NS) if split_edges else ROWS // NS
    iters = rpt // KR

    mesh = plsc.VectorSubcoreMesh(core_axis_name="c", subcore_axis_name="s",
                                  num_cores=NC, num_subcores=NS)

    @functools.partial(
        pl.kernel, mesh=mesh,
        compiler_params=pltpu.CompilerParams(use_tc_tiling_on_sc=False,
                                             internal_scratch_in_bytes=0),
        out_type=jax.ShapeDtypeStruct((n_out * NP, CW), jnp.float32),
        scratch_types=[
            pltpu.VMEM((KR, 128), jnp.int32),
            pltpu.VMEM((KR, 128), jnp.int32),
            pltpu.VMEM((KR * 128, CW), jnp.float32),
            pltpu.VMEM_SHARED((NP, CW), jnp.float32),
            pltpu.SemaphoreType.DMA,
            pltpu.SemaphoreType.DMA,
        ],
    )
    def k(src_h, dst_h, y_h, z_h, idxs_v, idxd_v, rows_v, zsp, gsem, ssem):
        c = lax.axis_index("c")
        s = lax.axis_index("s")
        nrows = KR * 128

        def fill(val):
            def frow(i, _):
                rows_v[i, :] = jnp.full((CW,), val, jnp.float32)
                return 0
            lax.fori_loop(0, nrows, frow, 0)

        for cc in range(tpc):
            chunk = c * tpc + cc
            toff = chunk * (N + 1)
            fill(0.0)
            for zz in range(TPT // nrows):
                pltpu.sync_copy(rows_v,
                                zsp.at[pl.ds(s * TPT + zz * nrows, nrows)])
            rem = TPT % nrows
            if rem:
                pltpu.sync_copy(
                    rows_v.at[pl.ds(0, rem)],
                    zsp.at[pl.ds(s * TPT + (TPT // nrows) * nrows, rem)])
            if no_gather:
                fill(1.0)
            plsc.subcore_barrier()

            base = (s * NC + c if split_edges else s) * rpt

            def it_body(it, _):
                r0 = base + it * KR
                pltpu.sync_copy(dst_h.at[pl.ds(r0, KR)], idxd_v)
                if not no_gather:
                    pltpu.sync_copy(src_h.at[pl.ds(r0, KR)], idxs_v)
                    if n_tables > 1:
                        for j in range(KR):
                            for v in range(8):
                                cur = idxs_v[j, pl.ds(v * 16, 16)]
                                idxs_v[j, pl.ds(v * 16, 16)] = cur + toff
                    descs = [
                        pltpu.async_copy(y_h.at[idxs_v.at[j]],
                                         rows_v.at[pl.ds(j * 128, 128)], gsem)
                        for j in range(KR)
                    ]
                    for d in descs:
                        d.wait()
                sdescs = [
                    pltpu.async_copy(rows_v.at[pl.ds(j * 128, 128)],
                                     zsp.at[idxd_v.at[j]], ssem, add=True)
                    for j in range(KR)
                ]
                for d in sdescs:
                    d.wait()
                return 0

            lax.fori_loop(0, iters, it_body, 0)
            plsc.subcore_barrier()
            ooff = (c if split_edges else chunk) * NP
            pltpu.sync_copy(zsp.at[pl.ds(s * TPT, TPT)],
                            z_h.at[pl.ds(ooff + s * TPT, TPT)])
            if cc + 1 < tpc:
                plsc.subcore_barrier()

    return k


def _full(shape):
    return pl.BlockSpec(shape, lambda i: tuple(0 for _ in shape))


def _rows(w):
    return pl.BlockSpec((BLK, w), lambda i: (i, 0))


def _chunks():
    return pl.BlockSpec((4, BLK, CW), lambda i: (0, i, 0))


def _tc1(zda, zdb, x):
    def body(za_ref, zb_ref, x_ref, dinv_ref, y_ref):
        deg = za_ref[:, :1] + zb_ref[:, :1] + 1.0
        dinv = lax.rsqrt(deg)
        dinv_ref[...] = dinv
        y_ref[...] = jnp.concatenate(
            [dinv * x_ref[...], jnp.zeros((BLK, CW - 4), jnp.float32)], axis=1)

    return pl.pallas_call(
        body,
        grid=(GRID,),
        in_specs=[_rows(CW), _rows(CW), _rows(4)],
        out_specs=[_rows(1), _rows(CW)],
        out_shape=[jax.ShapeDtypeStruct((N, 1), jnp.float32),
                   jax.ShapeDtypeStruct((N + 8, CW), jnp.float32)],
    )(zda, zdb, x)


def _tc2(z1a, z1b, y1p, dinv, W1, b1):
    def body(za_ref, zb_ref, y_ref, dinv_ref, w_ref, b_ref, out_ref):
        dinv = dinv_ref[...]
        u1 = dinv * (za_ref[...] + zb_ref[...] + y_ref[...])[:, :4]
        h1 = jax.nn.relu(
            lax.dot_general(u1, w_ref[...], (((1,), (0,)), ((), ())),
                            preferred_element_type=jnp.float32) + b_ref[...])
        y2 = dinv * h1
        for cix in range(4):
            out_ref[cix, :, :] = y2[:, cix * CW:(cix + 1) * CW]

    return pl.pallas_call(
        body,
        grid=(GRID,),
        in_specs=[_rows(CW), _rows(CW), _rows(CW), _rows(1),
                  _full((4, HID)), _full((1, HID))],
        out_specs=_chunks(),
        out_shape=jax.ShapeDtypeStruct((4, N + 1, CW), jnp.float32),
    )(z1a, z1b, y1p[:N], dinv, W1, b1.reshape(1, HID))


def _tc3(z2, y2, dinv, W2, b2):
    def body(z_ref, y_ref, dinv_ref, w_ref, b_ref, out_ref):
        dinv = dinv_ref[...]
        zfull = jnp.concatenate(
            [z_ref[cix] + y_ref[cix] for cix in range(4)], axis=1)
        u2 = dinv * zfull
        h2 = jax.nn.relu(
            lax.dot_general(u2, w_ref[...], (((1,), (0,)), ((), ())),
                            preferred_element_type=jnp.float32) + b_ref[...])
        y3 = dinv * h2
        for cix in range(4):
            out_ref[cix, :, :] = y3[:, cix * CW:(cix + 1) * CW]

    return pl.pallas_call(
        body,
        grid=(GRID,),
        in_specs=[pl.BlockSpec((4, BLK, CW), lambda i: (0, i, 0)),
                  pl.BlockSpec((4, BLK, CW), lambda i: (0, i, 0)),
                  _rows(1), _full((HID, HID)), _full((1, HID))],
        out_specs=_chunks(),
        out_shape=jax.ShapeDtypeStruct((4, N + 1, CW), jnp.float32),
    )(z2, y2[:, :N], dinv, W2, b2.reshape(1, HID))


def _tc4(z3, y3, dinv, batch3, W3, b3, lin_W, lin_b):
    def body(z_ref, y_ref, dinv_ref, b_ref, w3_ref, b3_ref, lw_ref, lb_ref,
             out_ref, acc_ref):
        i = pl.program_id(0)

        @pl.when(i == 0)
        def _():
            acc_ref[...] = jnp.zeros((G, HID + 1), jnp.float32)

        dinv = dinv_ref[...]
        u3 = dinv * jnp.concatenate(
            [z_ref[cix] + y_ref[cix] for cix in range(4)], axis=1)
        ext = jnp.concatenate([u3, jnp.ones((BLK, 1), jnp.float32)], axis=1)
        bvals = b_ref[0, 0, :]
        onehot = (bvals[:, None]
                  == lax.broadcasted_iota(jnp.int32, (BLK, G), 1)
                  ).astype(jnp.float32)
        acc_ref[...] += lax.dot_general(
            onehot, ext, (((0,), (0,)), ((), ())),
            preferred_element_type=jnp.float32)

        @pl.when(i == GRID - 1)
        def _():
            acc = acc_ref[...]
            cnt = acc[:, HID:HID + 1]
            pooled = acc[:, :HID] / jnp.maximum(cnt, 1.0)
            g3 = lax.dot_general(pooled, w3_ref[...], (((1,), (0,)), ((), ())),
                                 preferred_element_type=jnp.float32)
            g3 = g3 + b3_ref[...] * (cnt > 0).astype(jnp.float32)
            out_ref[...] = lax.dot_general(
                g3, lw_ref[...], (((1,), (0,)), ((), ())),
                preferred_element_type=jnp.float32) + lb_ref[...]

    return pl.pallas_call(
        body,
        grid=(GRID,),
        in_specs=[pl.BlockSpec((4, BLK, CW), lambda i: (0, i, 0)),
                  pl.BlockSpec((4, BLK, CW), lambda i: (0, i, 0)),
                  _rows(1),
                  pl.BlockSpec((1, 1, BLK), lambda i: (i, 0, 0)),
                  _full((HID, HID)), _full((1, HID)),
                  _full((HID, 3)), _full((1, 3))],
        out_specs=pl.BlockSpec((G, 3), lambda i: (0, 0)),
        out_shape=jax.ShapeDtypeStruct((G, 3), jnp.float32),
        scratch_shapes=[pltpu.VMEM((G, HID + 1), jnp.float32)],
    )(z3, y3[:, :N], dinv, batch3, W3, b3.reshape(1, HID),
      lin_W, lin_b.reshape(1, 3))


def kernel(x, edge_index, batch, W1, b1, W2, b2, W3, b3, lin_W, lin_b):
    padv = jnp.full((E_PAD - E,), N, jnp.int32)
    srcp = jnp.concatenate([edge_index[0], padv]).reshape(ROWS, 128)
    dstp = jnp.concatenate([edge_index[1], padv]).reshape(ROWS, 128)
    dummy_y = jnp.zeros((8, CW), jnp.float32)

    zdeg = _sc_pass(1, True, True)(srcp, dstp, dummy_y)
    dinv, y1p = _tc1(zdeg[:N], zdeg[NP:NP + N], x)

    z1 = _sc_pass(1, True, False)(srcp, dstp, y1p)
    y2 = _tc2(z1[:N], z1[NP:NP + N], y1p, dinv, W1, b1)

    z2 = _sc_pass(4, False, False)(srcp, dstp, y2.reshape(4 * (N + 1), CW))
    y3 = _tc3(z2.reshape(4, NP, CW)[:, :N], y2, dinv, W2, b2)

    z3 = _sc_pass(4, False, False)(srcp, dstp, y3.reshape(4 * (N + 1), CW))
    batch3 = batch.reshape(GRID, 1, BLK)
    return _tc4(z3.reshape(4, NP, CW)[:, :N], y3, dinv, batch3,
                W3, b3, lin_W, lin_b)


# speedup vs baseline: 11.3297x; 1.0682x over previous
import functools

import jax
import jax.numpy as jnp
from jax import lax
from jax.experimental import pallas as pl
from jax.experimental.pallas import tpu as pltpu
from jax.experimental.pallas import tpu_sc as plsc

N = 100000
E = 3200000
G = 128
HID = 64
CW = 16
NC, NS = 2, 16
KR = 8
E_PAD = 3211264
ROWS = E_PAD // 128
NP = 102400
TPT = NP // NS
BLK = 1000
GRID = N // BLK


@functools.lru_cache(maxsize=None)
def _sc_pass(n_tables: int, split_edges: bool, no_gather: bool):
    tpc = 1 if split_edges else n_tables
    n_out = NC if split_edges else n_tables
    rpt = ROWS // (NC * NS) if split_edges else ROWS // NS
    iters = rpt // KR

    mesh = plsc.VectorSubcoreMesh(core_axis_name="c", subcore_axis_name="s",
                                  num_cores=NC, num_subcores=NS)

    @functools.partial(
        pl.kernel, mesh=mesh,
        compiler_params=pltpu.CompilerParams(use_tc_tiling_on_sc=False,
                                             internal_scratch_in_bytes=0),
        out_type=jax.ShapeDtypeStruct((n_out * NP, CW), jnp.float32),
        scratch_types=[
            pltpu.VMEM((KR, 128), jnp.int32),
            pltpu.VMEM((KR, 128), jnp.int32),
            pltpu.VMEM((KR * 128, CW), jnp.float32),
            pltpu.VMEM_SHARED((NP, CW), jnp.float32),
            pltpu.SemaphoreType.DMA,
            pltpu.SemaphoreType.DMA,
            pltpu.SemaphoreType.DMA,
        ],
    )
    def k(src_h, dst_h, y_h, z_h, idxs_v, idxd_v, rows_v, zsp,
          gsem, ssem, isem):
        c = lax.axis_index("c")
        s = lax.axis_index("s")
        nrows = KR * 128

        def fill(val):
            def frow(i, _):
                rows_v[i, :] = jnp.full((CW,), val, jnp.float32)
                return 0
            lax.fori_loop(0, nrows, frow, 0)

        for cc in range(tpc):
            chunk = c * tpc + cc
            toff = chunk * (N + 1)
            fill(0.0)
            for zz in range(TPT // nrows):
                pltpu.sync_copy(rows_v,
                                zsp.at[pl.ds(s * TPT + zz * nrows, nrows)])
            rem = TPT % nrows
            if rem:
                pltpu.sync_copy(
                    rows_v.at[pl.ds(0, rem)],
                    zsp.at[pl.ds(s * TPT + (TPT // nrows) * nrows, rem)])
            if no_gather:
                fill(1.0)
            plsc.subcore_barrier()

            base = (s * NC + c if split_edges else s) * rpt
            HKR = KR // 2
            S = rpt // HKR

            def issue_idx(st, par):
                r1 = base + st * HKR
                pltpu.async_copy(dst_h.at[pl.ds(r1, HKR)],
                                 idxd_v.at[pl.ds(par * HKR, HKR)], isem)
                if not no_gather:
                    pltpu.async_copy(src_h.at[pl.ds(r1, HKR)],
                                     idxs_v.at[pl.ds(par * HKR, HKR)], isem)

            def drain_scatters(par):
                for jj in range(HKR):
                    j = par * HKR + jj
                    pltpu.make_async_copy(
                        rows_v.at[pl.ds(j * 128, 128)],
                        zsp.at[idxd_v.at[j]], ssem).wait()

            def stage(st, p):
                r0 = base + st * HKR
                q = 1 - p
                pltpu.make_async_copy(
                    dst_h.at[pl.ds(r0, HKR)],
                    idxd_v.at[pl.ds(p * HKR, HKR)], isem).wait()
                if not no_gather:
                    pltpu.make_async_copy(
                        src_h.at[pl.ds(r0, HKR)],
                        idxs_v.at[pl.ds(p * HKR, HKR)], isem).wait()

                @pl.when(st > 0)
                def _():
                    drain_scatters(q)

                @pl.when(st < S - 1)
                def _():
                    issue_idx(st + 1, q)

                if (not no_gather) and n_tables > 1:
                    for jj in range(HKR):
                        j = p * HKR + jj
                        for v in range(8):
                            cur = idxs_v[j, pl.ds(v * 16, 16)]
                            idxs_v[j, pl.ds(v * 16, 16)] = cur + toff
                if not no_gather:
                    descs = [
                        pltpu.async_copy(
                            y_h.at[idxs_v.at[p * HKR + jj]],
                            rows_v.at[pl.ds((p * HKR + jj) * 128, 128)], gsem)
                        for jj in range(HKR)
                    ]
                    for d in descs:
                        d.wait()
                for jj in range(HKR):
                    j = p * HKR + jj
                    pltpu.async_copy(rows_v.at[pl.ds(j * 128, 128)],
                                     zsp.at[idxd_v.at[j]], ssem, add=True)

            issue_idx(0, 0)

            def it_body(it, _):
                stage(2 * it, 0)
                stage(2 * it + 1, 1)
                return 0

            lax.fori_loop(0, S // 2, it_body, 0)
            drain_scatters(1)
            plsc.subcore_barrier()
            ooff = (c if split_edges else chunk) * NP
            pltpu.sync_copy(zsp.at[pl.ds(s * TPT, TPT)],
                            z_h.at[pl.ds(ooff + s * TPT, TPT)])
            if cc + 1 < tpc:
                plsc.subcore_barrier()

    return k


def _full(shape):
    return pl.BlockSpec(shape, lambda i: tuple(0 for _ in shape))


def _rows(w):
    return pl.BlockSpec((BLK, w), lambda i: (i, 0))


def _chunks():
    return pl.BlockSpec((4, BLK, CW), lambda i: (0, i, 0))


def _tc1(zda, zdb, x):
    def body(za_ref, zb_ref, x_ref, dinv_ref, y_ref):
        deg = za_ref[:, :1] + zb_ref[:, :1] + 1.0
        dinv = lax.rsqrt(deg)
        dinv_ref[...] = dinv
        y_ref[...] = jnp.concatenate(
            [dinv * x_ref[...], jnp.zeros((BLK, CW - 4), jnp.float32)], axis=1)

    return pl.pallas_call(
        body,
        grid=(GRID,),
        in_specs=[_rows(CW), _rows(CW), _rows(4)],
        out_specs=[_rows(1), _rows(CW)],
        out_shape=[jax.ShapeDtypeStruct((N, 1), jnp.float32),
                   jax.ShapeDtypeStruct((N + 8, CW), jnp.float32)],
    )(zda, zdb, x)


def _tc2(z1a, z1b, y1p, dinv, W1, b1):
    def body(za_ref, zb_ref, y_ref, dinv_ref, w_ref, b_ref, out_ref):
        dinv = dinv_ref[...]
        u1 = dinv * (za_ref[...] + zb_ref[...] + y_ref[...])[:, :4]
        h1 = jax.nn.relu(
            lax.dot_general(u1, w_ref[...], (((1,), (0,)), ((), ())),
                            preferred_element_type=jnp.float32) + b_ref[...])
        y2 = dinv * h1
        for cix in range(4):
            out_ref[cix, :, :] = y2[:, cix * CW:(cix + 1) * CW]

    return pl.pallas_call(
        body,
        grid=(GRID,),
        in_specs=[_rows(CW), _rows(CW), _rows(CW), _rows(1),
                  _full((4, HID)), _full((1, HID))],
        out_specs=_chunks(),
        out_shape=jax.ShapeDtypeStruct((4, N + 1, CW), jnp.float32),
    )(z1a, z1b, y1p[:N], dinv, W1, b1.reshape(1, HID))


def _tc3(z2, y2, dinv, W2, b2):
    def body(z_ref, y_ref, dinv_ref, w_ref, b_ref, out_ref):
        dinv = dinv_ref[...]
        zfull = jnp.concatenate(
            [z_ref[cix] + y_ref[cix] for cix in range(4)], axis=1)
        u2 = dinv * zfull
        h2 = jax.nn.relu(
            lax.dot_general(u2, w_ref[...], (((1,), (0,)), ((), ())),
                            preferred_element_type=jnp.float32) + b_ref[...])
        y3 = dinv * h2
        for cix in range(4):
            out_ref[cix, :, :] = y3[:, cix * CW:(cix + 1) * CW]

    return pl.pallas_call(
        body,
        grid=(GRID,),
        in_specs=[pl.BlockSpec((4, BLK, CW), lambda i: (0, i, 0)),
                  pl.BlockSpec((4, BLK, CW), lambda i: (0, i, 0)),
                  _rows(1), _full((HID, HID)), _full((1, HID))],
        out_specs=_chunks(),
        out_shape=jax.ShapeDtypeStruct((4, N + 1, CW), jnp.float32),
    )(z2, y2[:, :N], dinv, W2, b2.reshape(1, HID))


def _tc4(z3, y3, dinv, batch3, W3, b3, lin_W, lin_b):
    def body(z_ref, y_ref, dinv_ref, b_ref, w3_ref, b3_ref, lw_ref, lb_ref,
             out_ref, acc_ref):
        i = pl.program_id(0)

        @pl.when(i == 0)
        def _():
            acc_ref[...] = jnp.zeros((G, HID + 1), jnp.float32)

        dinv = dinv_ref[...]
        u3 = dinv * jnp.concatenate(
            [z_ref[cix] + y_ref[cix] for cix in range(4)], axis=1)
        ext = jnp.concatenate([u3, jnp.ones((BLK, 1), jnp.float32)], axis=1)
        bvals = b_ref[0, 0, :]
        onehot = (bvals[:, None]
                  == lax.broadcasted_iota(jnp.int32, (BLK, G), 1)
                  ).astype(jnp.float32)
        acc_ref[...] += lax.dot_general(
            onehot, ext, (((0,), (0,)), ((), ())),
            preferred_element_type=jnp.float32)

        @pl.when(i == GRID - 1)
        def _():
            acc = acc_ref[...]
            cnt = acc[:, HID:HID + 1]
            pooled = acc[:, :HID] / jnp.maximum(cnt, 1.0)
            g3 = lax.dot_general(pooled, w3_ref[...], (((1,), (0,)), ((), ())),
                                 preferred_element_type=jnp.float32)
            g3 = g3 + b3_ref[...] * (cnt > 0).astype(jnp.float32)
            out_ref[...] = lax.dot_general(
                g3, lw_ref[...], (((1,), (0,)), ((), ())),
                preferred_element_type=jnp.float32) + lb_ref[...]

    return pl.pallas_call(
        body,
        grid=(GRID,),
        in_specs=[pl.BlockSpec((4, BLK, CW), lambda i: (0, i, 0)),
                  pl.BlockSpec((4, BLK, CW), lambda i: (0, i, 0)),
                  _rows(1),
                  pl.BlockSpec((1, 1, BLK), lambda i: (i, 0, 0)),
                  _full((HID, HID)), _full((1, HID)),
                  _full((HID, 3)), _full((1, 3))],
        out_specs=pl.BlockSpec((G, 3), lambda i: (0, 0)),
        out_shape=jax.ShapeDtypeStruct((G, 3), jnp.float32),
        scratch_shapes=[pltpu.VMEM((G, HID + 1), jnp.float32)],
    )(z3, y3[:, :N], dinv, batch3, W3, b3.reshape(1, HID),
      lin_W, lin_b.reshape(1, 3))


def kernel(x, edge_index, batch, W1, b1, W2, b2, W3, b3, lin_W, lin_b):
    padv = jnp.full((E_PAD - E,), N, jnp.int32)
    srcp = jnp.concatenate([edge_index[0], padv]).reshape(ROWS, 128)
    dstp = jnp.concatenate([edge_index[1], padv]).reshape(ROWS, 128)
    dummy_y = jnp.zeros((8, CW), jnp.float32)

    zdeg = _sc_pass(1, True, True)(srcp, dstp, dummy_y)
    dinv, y1p = _tc1(zdeg[:N], zdeg[NP:NP + N], x)

    z1 = _sc_pass(1, True, False)(srcp, dstp, y1p)
    y2 = _tc2(z1[:N], z1[NP:NP + N], y1p, dinv, W1, b1)

    z2 = _sc_pass(4, False, False)(srcp, dstp, y2.reshape(4 * (N + 1), CW))
    y3 = _tc3(z2.reshape(4, NP, CW)[:, :N], y2, dinv, W2, b2)

    z3 = _sc_pass(4, False, False)(srcp, dstp, y3.reshape(4 * (N + 1), CW))
    batch3 = batch.reshape(GRID, 1, BLK)
    return _tc4(z3.reshape(4, NP, CW)[:, :N], y3, dinv, batch3,
                W3, b3, lin_W, lin_b)


# speedup vs baseline: 12.1413x; 1.0716x over previous
import functools

import jax
import jax.numpy as jnp
from jax import lax
from jax.experimental import pallas as pl
from jax.experimental.pallas import tpu as pltpu
from jax.experimental.pallas import tpu_sc as plsc

N = 100000
E = 3200000
G = 128
HID = 64
CW = 16
NC, NS = 2, 16
KR = 8
E_PAD = 3211264
ROWS = E_PAD // 128
NP = 102400
TPT = NP // NS
BLK = 1000
GRID = N // BLK


@functools.lru_cache(maxsize=None)
def _sc_pass(n_tables: int, split_edges: bool, no_gather: bool):
    tpc = 1 if split_edges else n_tables
    n_out = NC if split_edges else n_tables
    rpt = ROWS // (NC * NS) if split_edges else ROWS // NS
    iters = rpt // KR

    mesh = plsc.VectorSubcoreMesh(core_axis_name="c", subcore_axis_name="s",
                                  num_cores=NC, num_subcores=NS)

    @functools.partial(
        pl.kernel, mesh=mesh,
        compiler_params=pltpu.CompilerParams(use_tc_tiling_on_sc=False,
                                             internal_scratch_in_bytes=0),
        out_type=jax.ShapeDtypeStruct((n_out * NP, CW), jnp.float32),
        scratch_types=[
            pltpu.VMEM((KR, 128), jnp.int32),
            pltpu.VMEM((KR, 128), jnp.int32),
            pltpu.VMEM((KR * 128, CW), jnp.float32),
            pltpu.VMEM_SHARED((NP, CW), jnp.float32),
            pltpu.SemaphoreType.DMA,
            pltpu.SemaphoreType.DMA,
            pltpu.SemaphoreType.DMA,
        ],
    )
    def k(src_h, dst_h, y_h, z_h, idxs_v, idxd_v, rows_v, zsp,
          gsem, ssem, isem):
        c = lax.axis_index("c")
        s = lax.axis_index("s")
        nrows = KR * 128

        def fill(val):
            def frow(i, _):
                rows_v[i, :] = jnp.full((CW,), val, jnp.float32)
                return 0
            lax.fori_loop(0, nrows, frow, 0)

        for cc in range(tpc):
            chunk = c * tpc + cc
            toff = chunk * (N + 1)
            fill(0.0)
            for zz in range(TPT // nrows):
                pltpu.sync_copy(rows_v,
                                zsp.at[pl.ds(s * TPT + zz * nrows, nrows)])
            rem = TPT % nrows
            if rem:
                pltpu.sync_copy(
                    rows_v.at[pl.ds(0, rem)],
                    zsp.at[pl.ds(s * TPT + (TPT // nrows) * nrows, rem)])
            if no_gather:
                fill(1.0)
            plsc.subcore_barrier()

            base = (s * NC + c if split_edges else s) * rpt
            HKR = KR // 2
            S = rpt // HKR

            def issue_idx(st, par):
                r1 = base + st * HKR
                pltpu.async_copy(dst_h.at[pl.ds(r1, HKR)],
                                 idxd_v.at[pl.ds(par * HKR, HKR)], isem)
                if not no_gather:
                    pltpu.async_copy(src_h.at[pl.ds(r1, HKR)],
                                     idxs_v.at[pl.ds(par * HKR, HKR)], isem)

            def drain_scatters(par):
                for jj in range(HKR):
                    j = par * HKR + jj
                    pltpu.make_async_copy(
                        rows_v.at[pl.ds(j * 128, 128)],
                        zsp.at[idxd_v.at[j]], ssem).wait()

            def stage(st, p):
                r0 = base + st * HKR
                q = 1 - p
                pltpu.make_async_copy(
                    dst_h.at[pl.ds(r0, HKR)],
                    idxd_v.at[pl.ds(p * HKR, HKR)], isem).wait()
                if not no_gather:
                    pltpu.make_async_copy(
                        src_h.at[pl.ds(r0, HKR)],
                        idxs_v.at[pl.ds(p * HKR, HKR)], isem).wait()

                @pl.when(st > 0)
                def _():
                    drain_scatters(q)

                @pl.when(st < S - 1)
                def _():
                    issue_idx(st + 1, q)

                if (not no_gather) and n_tables > 1:
                    for jj in range(HKR):
                        j = p * HKR + jj
                        for v in range(8):
                            cur = idxs_v[j, pl.ds(v * 16, 16)]
                            idxs_v[j, pl.ds(v * 16, 16)] = cur + toff
                if not no_gather:
                    descs = [
                        pltpu.async_copy(
                            y_h.at[idxs_v.at[p * HKR + jj]],
                            rows_v.at[pl.ds((p * HKR + jj) * 128, 128)], gsem)
                        for jj in range(HKR)
                    ]
                    for d in descs:
                        d.wait()
                for jj in range(HKR):
                    j = p * HKR + jj
                    pltpu.async_copy(rows_v.at[pl.ds(j * 128, 128)],
                                     zsp.at[idxd_v.at[j]], ssem, add=True)

            issue_idx(0, 0)

            def it_body(it, _):
                stage(2 * it, 0)
                stage(2 * it + 1, 1)
                return 0

            lax.fori_loop(0, S // 2, it_body, 0)
            drain_scatters(1)
            plsc.subcore_barrier()
            ooff = (c if split_edges else chunk) * NP
            pltpu.sync_copy(zsp.at[pl.ds(s * TPT, TPT)],
                            z_h.at[pl.ds(ooff + s * TPT, TPT)])
            if cc + 1 < tpc:
                plsc.subcore_barrier()

    return k


def _full(shape):
    return pl.BlockSpec(shape, lambda i: tuple(0 for _ in shape))


def _rows(w):
    return pl.BlockSpec((BLK, w), lambda i: (i, 0))


def _chunks():
    return pl.BlockSpec((4, BLK, CW), lambda i: (0, i, 0))


def _tc1(zdeg, x):
    def body(za_ref, zb_ref, x_ref, dinv_ref, y_ref):
        deg = za_ref[0, :, :1] + zb_ref[0, :, :1] + 1.0
        dinv = lax.rsqrt(deg)
        dinv_ref[...] = dinv
        y_ref[...] = jnp.concatenate(
            [dinv * x_ref[...], jnp.zeros((BLK, CW - 4), jnp.float32)], axis=1)

    return pl.pallas_call(
        body,
        grid=(GRID,),
        in_specs=[pl.BlockSpec((1, BLK, CW), lambda i: (0, i, 0)),
                  pl.BlockSpec((1, BLK, CW), lambda i: (1, i, 0)),
                  _rows(4)],
        out_specs=[_rows(1), _rows(CW)],
        out_shape=[jax.ShapeDtypeStruct((N, 1), jnp.float32),
                   jax.ShapeDtypeStruct((N + 8, CW), jnp.float32)],
    )(zdeg, zdeg, x)


def _tc2(z1, y1p, dinv, W1, b1):
    def body(za_ref, zb_ref, y_ref, dinv_ref, w_ref, b_ref, out_ref):
        dinv = dinv_ref[...]
        u1 = dinv * (za_ref[0] + zb_ref[0] + y_ref[...])[:, :4]
        h1 = jax.nn.relu(
            lax.dot_general(u1, w_ref[...], (((1,), (0,)), ((), ())),
                            preferred_element_type=jnp.float32) + b_ref[...])
        y2 = dinv * h1
        for cix in range(4):
            out_ref[cix, :, :] = y2[:, cix * CW:(cix + 1) * CW]

    return pl.pallas_call(
        body,
        grid=(GRID,),
        in_specs=[pl.BlockSpec((1, BLK, CW), lambda i: (0, i, 0)),
                  pl.BlockSpec((1, BLK, CW), lambda i: (1, i, 0)),
                  _rows(CW), _rows(1),
                  _full((4, HID)), _full((1, HID))],
        out_specs=_chunks(),
        out_shape=jax.ShapeDtypeStruct((4, N + 1, CW), jnp.float32),
    )(z1, z1, y1p, dinv, W1, b1.reshape(1, HID))


def _tc3(z2, y2, dinv, W2, b2):
    def body(z_ref, y_ref, dinv_ref, w_ref, b_ref, out_ref):
        dinv = dinv_ref[...]
        zfull = jnp.concatenate(
            [z_ref[cix] + y_ref[cix] for cix in range(4)], axis=1)
        u2 = dinv * zfull
        h2 = jax.nn.relu(
            lax.dot_general(u2, w_ref[...], (((1,), (0,)), ((), ())),
                            preferred_element_type=jnp.float32) + b_ref[...])
        y3 = dinv * h2
        for cix in range(4):
            out_ref[cix, :, :] = y3[:, cix * CW:(cix + 1) * CW]

    return pl.pallas_call(
        body,
        grid=(GRID,),
        in_specs=[pl.BlockSpec((4, BLK, CW), lambda i: (0, i, 0)),
                  pl.BlockSpec((4, BLK, CW), lambda i: (0, i, 0)),
                  _rows(1), _full((HID, HID)), _full((1, HID))],
        out_specs=_chunks(),
        out_shape=jax.ShapeDtypeStruct((4, N + 1, CW), jnp.float32),
    )(z2, y2, dinv, W2, b2.reshape(1, HID))


def _tc4(z3, y3, dinv, batch3, W3, b3, lin_W, lin_b):
    def body(z_ref, y_ref, dinv_ref, b_ref, w3_ref, b3_ref, lw_ref, lb_ref,
             out_ref, acc_ref):
        i = pl.program_id(0)

        @pl.when(i == 0)
        def _():
            acc_ref[...] = jnp.zeros((G, HID + 1), jnp.float32)

        dinv = dinv_ref[...]
        u3 = dinv * jnp.concatenate(
            [z_ref[cix] + y_ref[cix] for cix in range(4)], axis=1)
        ext = jnp.concatenate([u3, jnp.ones((BLK, 1), jnp.float32)], axis=1)
        bvals = b_ref[0, 0, :]
        onehot = (bvals[:, None]
                  == lax.broadcasted_iota(jnp.int32, (BLK, G), 1)
                  ).astype(jnp.float32)
        acc_ref[...] += lax.dot_general(
            onehot, ext, (((0,), (0,)), ((), ())),
            preferred_element_type=jnp.float32)

        @pl.when(i == GRID - 1)
        def _():
            acc = acc_ref[...]
            cnt = acc[:, HID:HID + 1]
            pooled = acc[:, :HID] / jnp.maximum(cnt, 1.0)
            g3 = lax.dot_general(pooled, w3_ref[...], (((1,), (0,)), ((), ())),
                                 preferred_element_type=jnp.float32)
            g3 = g3 + b3_ref[...] * (cnt > 0).astype(jnp.float32)
            out_ref[...] = lax.dot_general(
                g3, lw_ref[...], (((1,), (0,)), ((), ())),
                preferred_element_type=jnp.float32) + lb_ref[...]

    return pl.pallas_call(
        body,
        grid=(GRID,),
        in_specs=[pl.BlockSpec((4, BLK, CW), lambda i: (0, i, 0)),
                  pl.BlockSpec((4, BLK, CW), lambda i: (0, i, 0)),
                  _rows(1),
                  pl.BlockSpec((1, 1, BLK), lambda i: (i, 0, 0)),
                  _full((HID, HID)), _full((1, HID)),
                  _full((HID, 3)), _full((1, 3))],
        out_specs=pl.BlockSpec((G, 3), lambda i: (0, 0)),
        out_shape=jax.ShapeDtypeStruct((G, 3), jnp.float32),
        scratch_shapes=[pltpu.VMEM((G, HID + 1), jnp.float32)],
    )(z3, y3, dinv, batch3, W3, b3.reshape(1, HID),
      lin_W, lin_b.reshape(1, 3))


def kernel(x, edge_index, batch, W1, b1, W2, b2, W3, b3, lin_W, lin_b):
    padv = jnp.full((E_PAD - E,), N, jnp.int32)
    srcp = jnp.concatenate([edge_index[0], padv]).reshape(ROWS, 128)
    dstp = jnp.concatenate([edge_index[1], padv]).reshape(ROWS, 128)
    dummy_y = jnp.zeros((8, CW), jnp.float32)

    zdeg = _sc_pass(1, True, True)(srcp, dstp, dummy_y)
    dinv, y1p = _tc1(zdeg.reshape(2, NP, CW), x)

    z1 = _sc_pass(1, True, False)(srcp, dstp, y1p)
    y2 = _tc2(z1.reshape(2, NP, CW), y1p, dinv, W1, b1)

    z2 = _sc_pass(4, False, False)(srcp, dstp, y2.reshape(4 * (N + 1), CW))
    y3 = _tc3(z2.reshape(4, NP, CW), y2, dinv, W2, b2)

    z3 = _sc_pass(4, False, False)(srcp, dstp, y3.reshape(4 * (N + 1), CW))
    batch3 = batch.reshape(GRID, 1, BLK)
    return _tc4(z3.reshape(4, NP, CW), y3, dinv, batch3,
                W3, b3, lin_W, lin_b)


# speedup vs baseline: 12.1468x; 1.0004x over previous
import functools

import jax
import jax.numpy as jnp
from jax import lax
from jax.experimental import pallas as pl
from jax.experimental.pallas import tpu as pltpu
from jax.experimental.pallas import tpu_sc as plsc

N = 100000
E = 3200000
G = 128
HID = 64
CW = 16
NC, NS = 2, 16
KR = 8
E_PAD = 3211264
ROWS = E_PAD // 128
NP = 102400
TPT = NP // NS
BLK = 1000
GRID = N // BLK


@functools.lru_cache(maxsize=None)
def _sc_pass(n_tables: int, split_edges: bool, no_gather: bool):
    tpc = 1 if split_edges else n_tables
    n_out = NC if split_edges else n_tables
    rpt = ROWS // (NC * NS) if split_edges else ROWS // NS

    mesh = plsc.VectorSubcoreMesh(core_axis_name="c", subcore_axis_name="s",
                                  num_cores=NC, num_subcores=NS)

    @functools.partial(
        pl.kernel, mesh=mesh,
        compiler_params=pltpu.CompilerParams(use_tc_tiling_on_sc=False,
                                             internal_scratch_in_bytes=0),
        out_type=jax.ShapeDtypeStruct((n_out * NP, CW), jnp.float32),
        scratch_types=[
            pltpu.VMEM((KR, 128), jnp.int32),
            pltpu.VMEM((KR, 128), jnp.int32),
            pltpu.VMEM((KR * 128, CW), jnp.float32),
            pltpu.VMEM_SHARED((NP, CW), jnp.float32),
            pltpu.SemaphoreType.DMA,
            pltpu.SemaphoreType.DMA,
            pltpu.SemaphoreType.DMA,
        ],
    )
    def k(src_h, dst_h, y_h, z_h, idxs_v, idxd_v, rows_v, zsp,
          gsem, ssem, isem):
        c = lax.axis_index("c")
        s = lax.axis_index("s")
        nrows = KR * 128

        def fill(val):
            def frow(i, _):
                rows_v[i, :] = jnp.full((CW,), val, jnp.float32)
                return 0
            lax.fori_loop(0, nrows, frow, 0)

        for cc in range(tpc):
            chunk = c * tpc + cc
            toff = chunk * (N + 1)
            fill(0.0)
            for zz in range(TPT // nrows):
                pltpu.sync_copy(rows_v,
                                zsp.at[pl.ds(s * TPT + zz * nrows, nrows)])
            rem = TPT % nrows
            if rem:
                pltpu.sync_copy(
                    rows_v.at[pl.ds(0, rem)],
                    zsp.at[pl.ds(s * TPT + (TPT // nrows) * nrows, rem)])
            if no_gather:
                fill(1.0)
            plsc.subcore_barrier()

            base = (s * NC + c if split_edges else s) * rpt
            HKR = KR // 2
            S = rpt // HKR

            def issue_idx(st, par):
                r1 = base + st * HKR
                pltpu.async_copy(dst_h.at[pl.ds(r1, HKR)],
                                 idxd_v.at[pl.ds(par * HKR, HKR)], isem)
                if not no_gather:
                    pltpu.async_copy(src_h.at[pl.ds(r1, HKR)],
                                     idxs_v.at[pl.ds(par * HKR, HKR)], isem)

            def drain_scatters(par):
                for jj in range(HKR):
                    j = par * HKR + jj
                    pltpu.make_async_copy(
                        rows_v.at[pl.ds(j * 128, 128)],
                        zsp.at[idxd_v.at[j]], ssem).wait()

            def stage(st, p):
                r0 = base + st * HKR
                q = 1 - p
                pltpu.make_async_copy(
                    dst_h.at[pl.ds(r0, HKR)],
                    idxd_v.at[pl.ds(p * HKR, HKR)], isem).wait()
                if not no_gather:
                    pltpu.make_async_copy(
                        src_h.at[pl.ds(r0, HKR)],
                        idxs_v.at[pl.ds(p * HKR, HKR)], isem).wait()

                @pl.when(st > 0)
                def _():
                    drain_scatters(q)

                @pl.when(st < S - 1)
                def _():
                    issue_idx(st + 1, q)

                if (not no_gather) and n_tables > 1:
                    for jj in range(HKR):
                        j = p * HKR + jj
                        for v in range(8):
                            cur = idxs_v[j, pl.ds(v * 16, 16)]
                            idxs_v[j, pl.ds(v * 16, 16)] = cur + toff
                if not no_gather:
                    descs = [
                        pltpu.async_copy(
                            y_h.at[idxs_v.at[p * HKR + jj]],
                            rows_v.at[pl.ds((p * HKR + jj) * 128, 128)], gsem)
                        for jj in range(HKR)
                    ]
                    for d in descs:
                        d.wait()
                for jj in range(HKR):
                    j = p * HKR + jj
                    pltpu.async_copy(rows_v.at[pl.ds(j * 128, 128)],
                                     zsp.at[idxd_v.at[j]], ssem, add=True)

            issue_idx(0, 0)

            def it_body(it, _):
                stage(2 * it, 0)
                stage(2 * it + 1, 1)
                return 0

            lax.fori_loop(0, S // 2, it_body, 0)
            drain_scatters(1)
            plsc.subcore_barrier()
            ooff = (c if split_edges else chunk) * NP
            pltpu.sync_copy(zsp.at[pl.ds(s * TPT, TPT)],
                            z_h.at[pl.ds(ooff + s * TPT, TPT)])
            if cc + 1 < tpc:
                plsc.subcore_barrier()

    return k


def _full(shape):
    return pl.BlockSpec(shape, lambda i: tuple(0 for _ in shape))


def _rows(w):
    return pl.BlockSpec((BLK, w), lambda i: (i, 0))


def _chunks():
    return pl.BlockSpec((4, BLK, CW), lambda i: (0, i, 0))


def _tc1(zdeg, x):
    def body(za_ref, zb_ref, x_ref, dinv_ref, y_ref):
        deg = za_ref[0, :, :1] + zb_ref[0, :, :1] + 1.0
        dinv = lax.rsqrt(deg)
        dinv_ref[...] = dinv
        y_ref[...] = jnp.concatenate(
            [dinv * x_ref[...], jnp.zeros((BLK, CW - 4), jnp.float32)], axis=1)

    return pl.pallas_call(
        body,
        grid=(GRID,),
        in_specs=[pl.BlockSpec((1, BLK, CW), lambda i: (0, i, 0)),
                  pl.BlockSpec((1, BLK, CW), lambda i: (1, i, 0)),
                  _rows(4)],
        out_specs=[_rows(1), _rows(CW)],
        out_shape=[jax.ShapeDtypeStruct((N, 1), jnp.float32),
                   jax.ShapeDtypeStruct((N + 8, CW), jnp.float32)],
    )(zdeg, zdeg, x)


def _tc2(z1, y1p, dinv, W1, b1):
    def body(za_ref, zb_ref, y_ref, dinv_ref, w_ref, b_ref, out_ref):
        dinv = dinv_ref[...]
        u1 = dinv * (za_ref[0] + zb_ref[0] + y_ref[...])[:, :4]
        h1 = jax.nn.relu(
            lax.dot_general(u1, w_ref[...], (((1,), (0,)), ((), ())),
                            preferred_element_type=jnp.float32) + b_ref[...])
        y2 = dinv * h1
        for cix in range(4):
            out_ref[cix, :, :] = y2[:, cix * CW:(cix + 1) * CW]

    return pl.pallas_call(
        body,
        grid=(GRID,),
        in_specs=[pl.BlockSpec((1, BLK, CW), lambda i: (0, i, 0)),
                  pl.BlockSpec((1, BLK, CW), lambda i: (1, i, 0)),
                  _rows(CW), _rows(1),
                  _full((4, HID)), _full((1, HID))],
        out_specs=_chunks(),
        out_shape=jax.ShapeDtypeStruct((4, N + 1, CW), jnp.float32),
    )(z1, z1, y1p, dinv, W1, b1.reshape(1, HID))


def _tc3(z2, y2, dinv, W2, b2):
    def body(z_ref, y_ref, dinv_ref, w_ref, b_ref, out_ref):
        dinv = dinv_ref[...]
        zfull = jnp.concatenate(
            [z_ref[cix] + y_ref[cix] for cix in range(4)], axis=1)
        u2 = dinv * zfull
        h2 = jax.nn.relu(
            lax.dot_general(u2, w_ref[...], (((1,), (0,)), ((), ())),
                            preferred_element_type=jnp.float32) + b_ref[...])
        y3 = dinv * h2
        for cix in range(4):
            out_ref[cix, :, :] = y3[:, cix * CW:(cix + 1) * CW]

    return pl.pallas_call(
        body,
        grid=(GRID,),
        in_specs=[pl.BlockSpec((4, BLK, CW), lambda i: (0, i, 0)),
                  pl.BlockSpec((4, BLK, CW), lambda i: (0, i, 0)),
                  _rows(1), _full((HID, HID)), _full((1, HID))],
        out_specs=_chunks(),
        out_shape=jax.ShapeDtypeStruct((4, N + 1, CW), jnp.float32),
    )(z2, y2, dinv, W2, b2.reshape(1, HID))


def _tc4(z3, y3, dinv, batch3, W3, b3, lin_W, lin_b):
    def body(z_ref, y_ref, dinv_ref, b_ref, w3_ref, b3_ref, lw_ref, lb_ref,
             out_ref, acc_ref):
        i = pl.program_id(0)

        @pl.when(i == 0)
        def _():
            acc_ref[...] = jnp.zeros((G, HID + 1), jnp.float32)

        dinv = dinv_ref[...]
        u3 = dinv * jnp.concatenate(
            [z_ref[cix] + y_ref[cix] for cix in range(4)], axis=1)
        ext = jnp.concatenate([u3, jnp.ones((BLK, 1), jnp.float32)], axis=1)
        bvals = b_ref[0, 0, :]
        onehot = (bvals[:, None]
                  == lax.broadcasted_iota(jnp.int32, (BLK, G), 1)
                  ).astype(jnp.float32)
        acc_ref[...] += lax.dot_general(
            onehot, ext, (((0,), (0,)), ((), ())),
            preferred_element_type=jnp.float32)

        @pl.when(i == GRID - 1)
        def _():
            acc = acc_ref[...]
            cnt = acc[:, HID:HID + 1]
            pooled = acc[:, :HID] / jnp.maximum(cnt, 1.0)
            g3 = lax.dot_general(pooled, w3_ref[...], (((1,), (0,)), ((), ())),
                                 preferred_element_type=jnp.float32)
            g3 = g3 + b3_ref[...] * (cnt > 0).astype(jnp.float32)
            out_ref[...] = lax.dot_general(
                g3, lw_ref[...], (((1,), (0,)), ((), ())),
                preferred_element_type=jnp.float32) + lb_ref[...]

    return pl.pallas_call(
        body,
        grid=(GRID,),
        in_specs=[pl.BlockSpec((4, BLK, CW), lambda i: (0, i, 0)),
                  pl.BlockSpec((4, BLK, CW), lambda i: (0, i, 0)),
                  _rows(1),
                  pl.BlockSpec((1, 1, BLK), lambda i: (i, 0, 0)),
                  _full((HID, HID)), _full((1, HID)),
                  _full((HID, 3)), _full((1, 3))],
        out_specs=pl.BlockSpec((G, 3), lambda i: (0, 0)),
        out_shape=jax.ShapeDtypeStruct((G, 3), jnp.float32),
        scratch_shapes=[pltpu.VMEM((G, HID + 1), jnp.float32)],
    )(z3, y3, dinv, batch3, W3, b3.reshape(1, HID),
      lin_W, lin_b.reshape(1, 3))


def kernel(x, edge_index, batch, W1, b1, W2, b2, W3, b3, lin_W, lin_b):
    padv = jnp.full((E_PAD - E,), N, jnp.int32)
    srcp = jnp.concatenate([edge_index[0], padv]).reshape(ROWS, 128)
    dstp = jnp.concatenate([edge_index[1], padv]).reshape(ROWS, 128)
    dummy_y = jnp.zeros((8, CW), jnp.float32)

    zdeg = _sc_pass(1, True, True)(srcp, dstp, dummy_y)
    dinv, y1p = _tc1(zdeg.reshape(2, NP, CW), x)

    z1 = _sc_pass(1, True, False)(srcp, dstp, y1p)
    y2 = _tc2(z1.reshape(2, NP, CW), y1p, dinv, W1, b1)

    z2 = _sc_pass(4, False, False)(srcp, dstp, y2.reshape(4 * (N + 1), CW))
    y3 = _tc3(z2.reshape(4, NP, CW), y2, dinv, W2, b2)

    z3 = _sc_pass(4, False, False)(srcp, dstp, y3.reshape(4 * (N + 1), CW))
    batch3 = batch.reshape(GRID, 1, BLK)
    return _tc4(z3.reshape(4, NP, CW), y3, dinv, batch3,
                W3, b3, lin_W, lin_b)
